# Initial kernel scaffold; baseline (speedup 1.0000x reference)
import jax
import jax.numpy as jnp
from jax.experimental import pallas as pl


def kernel(subword_embeddings, W_ih_f, W_hh_f, b_ih_f, b_hh_f, W_ih_b, W_hh_b, b_ih_b, b_hh_b, W_ffnn, b_ffnn, W_out, b_out):
    raise NotImplementedError("write your pallas kernel here")


# speedup vs baseline: 16.8141x; 16.8141x over previous
import jax
import jax.numpy as jnp
from jax.experimental import pallas as pl
from jax.experimental.pallas import tpu as pltpu

S = 512
D_IN = 1536
HID = 384
M = 30
G = 4 * HID
MPAD = 32


def _proj_kernel(x_ref, w_ref, b_ref, o_ref):
    o_ref[...] = (
        jnp.dot(x_ref[...], w_ref[...], preferred_element_type=jnp.float32)
        + b_ref[...]
    )


def _rec_span_kernel(xp_ref, whhf_ref, whhb_ref, w1_ref, w2_ref,
                     bffn_ref, wout_ref, bout_ref, out_ref,
                     emb_scr, bpad_scr):
    H = HID

    def cell(g, c):
        i = jax.nn.sigmoid(g[:, 0:H])
        f = jax.nn.sigmoid(g[:, H:2 * H])
        gg = jnp.tanh(g[:, 2 * H:3 * H])
        o = jax.nn.sigmoid(g[:, 3 * H:4 * H])
        c2 = f * c + i * gg
        h2 = o * jnp.tanh(c2)
        return h2, c2

    def step(t, carry):
        hf, cf, hb, cb = carry
        gf = xp_ref[pl.ds(t, 1), 0:G] + jnp.dot(
            hf, whhf_ref[...], preferred_element_type=jnp.float32)
        gb = xp_ref[pl.ds(S - 1 - t, 1), G:2 * G] + jnp.dot(
            hb, whhb_ref[...], preferred_element_type=jnp.float32)
        hf2, cf2 = cell(gf, cf)
        hb2, cb2 = cell(gb, cb)
        emb_scr[pl.ds(t, 1), 0:H] = hf2
        emb_scr[pl.ds(S - 1 - t, 1), H:2 * H] = hb2
        return hf2, cf2, hb2, cb2

    z = jnp.zeros((1, H), jnp.float32)
    jax.lax.fori_loop(0, S, step, (z, z, z, z))

    emb = emb_scr[...]
    A = jnp.dot(emb, w1_ref[...], preferred_element_type=jnp.float32) \
        + bffn_ref[...]
    B = jnp.dot(emb, w2_ref[...], preferred_element_type=jnp.float32)
    bpad_scr[0:S, :] = B
    bpad_scr[S:S + MPAD, :] = jnp.zeros((MPAD, HID), jnp.float32)
    rows = jax.lax.broadcasted_iota(jnp.int32, (S, 1), 0)
    for j in range(M):
        r = jnp.maximum(A + bpad_scr[j:j + S, :], 0.0)
        col = jnp.dot(r, wout_ref[...], preferred_element_type=jnp.float32) \
            + bout_ref[...]
        out_ref[:, j:j + 1] = jnp.where(rows + j < S, col, 0.0)


def kernel(subword_embeddings, W_ih_f, W_hh_f, b_ih_f, b_hh_f,
           W_ih_b, W_hh_b, b_ih_b, b_hh_b,
           W_ffnn, b_ffnn, W_out, b_out):
    f32 = jnp.float32
    Wcat = jnp.concatenate([W_ih_f.T, W_ih_b.T], axis=1)
    bias = jnp.concatenate([b_ih_f + b_hh_f, b_ih_b + b_hh_b])[None, :]

    CB = 768
    xp = pl.pallas_call(
        _proj_kernel,
        grid=(2 * G // CB,),
        in_specs=[
            pl.BlockSpec((S, D_IN), lambda i: (0, 0)),
            pl.BlockSpec((D_IN, CB), lambda i: (0, i)),
            pl.BlockSpec((1, CB), lambda i: (0, i)),
        ],
        out_specs=pl.BlockSpec((S, CB), lambda i: (0, i)),
        out_shape=jax.ShapeDtypeStruct((S, 2 * G), f32),
    )(subword_embeddings, Wcat, bias)

    out2d = pl.pallas_call(
        _rec_span_kernel,
        out_shape=jax.ShapeDtypeStruct((S, M), f32),
        scratch_shapes=[
            pltpu.VMEM((S, 2 * HID), f32),
            pltpu.VMEM((S + MPAD, HID), f32),
        ],
    )(xp, W_hh_f.T, W_hh_b.T,
      W_ffnn[:, :2 * HID].T, W_ffnn[:, 2 * HID:].T,
      b_ffnn[None, :], W_out.T, b_out[None, :])

    return out2d[:, :, None]


# speedup vs baseline: 20.3201x; 1.2085x over previous
import jax
import jax.numpy as jnp
from jax.experimental import pallas as pl
from jax.experimental.pallas import tpu as pltpu

S = 512
D_IN = 1536
HID = 384
M = 30
G = 4 * HID
MPAD = 32

_DNT = (((1,), (1,)), ((), ()))


def _proj_kernel(x_ref, wf_ref, wb_ref, bf_ref, bb_ref, of_ref, ob_ref):
    x = x_ref[...]
    of_ref[...] = jax.lax.dot_general(
        x, wf_ref[...], _DNT, preferred_element_type=jnp.float32) + bf_ref[...]
    ob_ref[...] = jax.lax.dot_general(
        x, wb_ref[...], _DNT, preferred_element_type=jnp.float32) + bb_ref[...]


def _rec_span_kernel(xpf_ref, xpb_ref, whhf_ref, whhb_ref, wffn_ref,
                     bffn_ref, wout_ref, bout_ref, out_ref,
                     emb_scr, bpad_scr, wtf_scr, wtb_scr):
    H = HID
    wtf_scr[...] = whhf_ref[...].T.astype(jnp.bfloat16)
    wtb_scr[...] = whhb_ref[...].T.astype(jnp.bfloat16)

    def cell(g, c):
        i = jax.nn.sigmoid(g[:, 0:H])
        f = jax.nn.sigmoid(g[:, H:2 * H])
        gg = jnp.tanh(g[:, 2 * H:3 * H])
        o = jax.nn.sigmoid(g[:, 3 * H:4 * H])
        c2 = f * c + i * gg
        h2 = o * jnp.tanh(c2)
        return h2, c2

    def step(t, carry):
        hf, cf, hb, cb = carry
        gf = xpf_ref[pl.ds(t, 1), :] + jnp.dot(
            hf.astype(jnp.bfloat16), wtf_scr[...],
            preferred_element_type=jnp.float32)
        gb = xpb_ref[pl.ds(S - 1 - t, 1), :] + jnp.dot(
            hb.astype(jnp.bfloat16), wtb_scr[...],
            preferred_element_type=jnp.float32)
        hf2, cf2 = cell(gf, cf)
        hb2, cb2 = cell(gb, cb)
        emb_scr[pl.ds(t, 1), 0:H] = hf2
        emb_scr[pl.ds(S - 1 - t, 1), H:2 * H] = hb2
        return hf2, cf2, hb2, cb2

    z = jnp.zeros((1, H), jnp.float32)
    jax.lax.fori_loop(0, S, step, (z, z, z, z))

    emb = emb_scr[...]
    A = jax.lax.dot_general(emb, wffn_ref[:, 0:2 * H], _DNT,
                            preferred_element_type=jnp.float32) + bffn_ref[...]
    B = jax.lax.dot_general(emb, wffn_ref[:, 2 * H:4 * H], _DNT,
                            preferred_element_type=jnp.float32)
    bpad_scr[0:S, :] = B
    bpad_scr[S:S + MPAD, :] = jnp.zeros((MPAD, HID), jnp.float32)
    rows = jax.lax.broadcasted_iota(jnp.int32, (S, 1), 0)
    for j in range(M):
        r = jnp.maximum(A + bpad_scr[j:j + S, :], 0.0)
        col = jnp.dot(r, wout_ref[...],
                      preferred_element_type=jnp.float32) + bout_ref[...]
        out_ref[:, j:j + 1] = jnp.where(rows + j < S, col, 0.0)


def kernel(subword_embeddings, W_ih_f, W_hh_f, b_ih_f, b_hh_f,
           W_ih_b, W_hh_b, b_ih_b, b_hh_b,
           W_ffnn, b_ffnn, W_out, b_out):
    f32 = jnp.float32
    bias_f = (b_ih_f + b_hh_f)[None, :]
    bias_b = (b_ih_b + b_hh_b)[None, :]

    RB = G // 2
    xpf, xpb = pl.pallas_call(
        _proj_kernel,
        grid=(G // RB,),
        in_specs=[
            pl.BlockSpec((S, D_IN), lambda i: (0, 0)),
            pl.BlockSpec((RB, D_IN), lambda i: (i, 0)),
            pl.BlockSpec((RB, D_IN), lambda i: (i, 0)),
            pl.BlockSpec((1, RB), lambda i: (0, i)),
            pl.BlockSpec((1, RB), lambda i: (0, i)),
        ],
        out_specs=[
            pl.BlockSpec((S, RB), lambda i: (0, i)),
            pl.BlockSpec((S, RB), lambda i: (0, i)),
        ],
        out_shape=[
            jax.ShapeDtypeStruct((S, G), f32),
            jax.ShapeDtypeStruct((S, G), f32),
        ],
    )(subword_embeddings, W_ih_f, W_ih_b, bias_f, bias_b)

    out2d = pl.pallas_call(
        _rec_span_kernel,
        out_shape=jax.ShapeDtypeStruct((S, M), f32),
        scratch_shapes=[
            pltpu.VMEM((S, 2 * HID), f32),
            pltpu.VMEM((S + MPAD, HID), f32),
            pltpu.VMEM((HID, G), jnp.bfloat16),
            pltpu.VMEM((HID, G), jnp.bfloat16),
        ],
    )(xpf, xpb, W_hh_f, W_hh_b, W_ffnn,
      b_ffnn[None, :], W_out.T, b_out[None, :])

    return out2d[:, :, None]


# speedup vs baseline: 23.4314x; 1.1531x over previous
import jax
import jax.numpy as jnp
from jax.experimental import pallas as pl
from jax.experimental.pallas import tpu as pltpu

S = 512
D_IN = 1536
HID = 384
M = 30
G = 4 * HID
MPAD = 32

_DNT = (((1,), (1,)), ((), ()))


def _proj_kernel(x_ref, wf_ref, wb_ref, bf_ref, bb_ref, of_ref, ob_ref):
    x = x_ref[...]
    of_ref[...] = jax.lax.dot_general(
        x, wf_ref[...], _DNT, preferred_element_type=jnp.float32) + bf_ref[...]
    ob_ref[...] = jax.lax.dot_general(
        x, wb_ref[...], _DNT, preferred_element_type=jnp.float32) + bb_ref[...]


def _rec_span_kernel(xpf_ref, xpb_ref, whhf_ref, whhb_ref, wffn_ref,
                     bffn_ref, wout_ref, bout_ref, out_ref,
                     emb_scr, bpad_scr, wtf_scr, wtb_scr):
    H = HID
    wtf_scr[...] = whhf_ref[...].T.astype(jnp.bfloat16)
    wtb_scr[...] = whhb_ref[...].T.astype(jnp.bfloat16)

    def cell(g, c):
        i = jax.nn.sigmoid(g[:, 0:H])
        f = jax.nn.sigmoid(g[:, H:2 * H])
        gg = jnp.tanh(g[:, 2 * H:3 * H])
        o = jax.nn.sigmoid(g[:, 3 * H:4 * H])
        c2 = f * c + i * gg
        h2 = o * jnp.tanh(c2)
        return h2, c2

    def step(t, carry):
        hf, cf, hb, cb = carry
        gf = xpf_ref[pl.ds(t, 1), :] + jnp.dot(
            hf.astype(jnp.bfloat16), wtf_scr[...],
            preferred_element_type=jnp.float32)
        gb = xpb_ref[pl.ds(S - 1 - t, 1), :] + jnp.dot(
            hb.astype(jnp.bfloat16), wtb_scr[...],
            preferred_element_type=jnp.float32)
        hf2, cf2 = cell(gf, cf)
        hb2, cb2 = cell(gb, cb)
        emb_scr[pl.ds(t, 1), 0:H] = hf2
        emb_scr[pl.ds(S - 1 - t, 1), H:2 * H] = hb2
        return hf2, cf2, hb2, cb2

    z = jnp.zeros((1, H), jnp.float32)
    jax.lax.fori_loop(0, S, step, (z, z, z, z), unroll=2)

    emb = emb_scr[...]
    A = jax.lax.dot_general(emb, wffn_ref[:, 0:2 * H], _DNT,
                            preferred_element_type=jnp.float32) + bffn_ref[...]
    B = jax.lax.dot_general(emb, wffn_ref[:, 2 * H:4 * H], _DNT,
                            preferred_element_type=jnp.float32)
    bpad_scr[0:S, :] = B
    bpad_scr[S:S + MPAD, :] = jnp.zeros((MPAD, HID), jnp.float32)
    rows = jax.lax.broadcasted_iota(jnp.int32, (S, 1), 0)
    for j in range(M):
        r = jnp.maximum(A + bpad_scr[j:j + S, :], 0.0)
        col = jnp.dot(r, wout_ref[...],
                      preferred_element_type=jnp.float32) + bout_ref[...]
        out_ref[:, j:j + 1] = jnp.where(rows + j < S, col, 0.0)


def kernel(subword_embeddings, W_ih_f, W_hh_f, b_ih_f, b_hh_f,
           W_ih_b, W_hh_b, b_ih_b, b_hh_b,
           W_ffnn, b_ffnn, W_out, b_out):
    f32 = jnp.float32
    bias_f = (b_ih_f + b_hh_f)[None, :]
    bias_b = (b_ih_b + b_hh_b)[None, :]

    RB = G // 2
    xpf, xpb = pl.pallas_call(
        _proj_kernel,
        grid=(G // RB,),
        in_specs=[
            pl.BlockSpec((S, D_IN), lambda i: (0, 0)),
            pl.BlockSpec((RB, D_IN), lambda i: (i, 0)),
            pl.BlockSpec((RB, D_IN), lambda i: (i, 0)),
            pl.BlockSpec((1, RB), lambda i: (0, i)),
            pl.BlockSpec((1, RB), lambda i: (0, i)),
        ],
        out_specs=[
            pl.BlockSpec((S, RB), lambda i: (0, i)),
            pl.BlockSpec((S, RB), lambda i: (0, i)),
        ],
        out_shape=[
            jax.ShapeDtypeStruct((S, G), f32),
            jax.ShapeDtypeStruct((S, G), f32),
        ],
    )(subword_embeddings, W_ih_f, W_ih_b, bias_f, bias_b)

    out2d = pl.pallas_call(
        _rec_span_kernel,
        out_shape=jax.ShapeDtypeStruct((S, M), f32),
        scratch_shapes=[
            pltpu.VMEM((S, 2 * HID), f32),
            pltpu.VMEM((S + MPAD, HID), f32),
            pltpu.VMEM((HID, G), jnp.bfloat16),
            pltpu.VMEM((HID, G), jnp.bfloat16),
        ],
    )(xpf, xpb, W_hh_f, W_hh_b, W_ffnn,
      b_ffnn[None, :], W_out.T, b_out[None, :])

    return out2d[:, :, None]


# speedup vs baseline: 25.4065x; 1.0843x over previous
import jax
import jax.numpy as jnp
from jax.experimental import pallas as pl
from jax.experimental.pallas import tpu as pltpu

S = 512
D_IN = 1536
HID = 384
M = 30
G = 4 * HID
MPAD = 32

_DNT = (((1,), (1,)), ((), ()))


def _proj_kernel(x_ref, wf_ref, wb_ref, bf_ref, bb_ref, of_ref, ob_ref):
    x = x_ref[...]
    of_ref[...] = jax.lax.dot_general(
        x, wf_ref[...], _DNT, preferred_element_type=jnp.float32) + bf_ref[...]
    ob_ref[...] = jax.lax.dot_general(
        x, wb_ref[...], _DNT, preferred_element_type=jnp.float32) + bb_ref[...]


def _rec_span_kernel(xpf_ref, xpb_ref, whhf_ref, whhb_ref, wffn_ref,
                     bffn_ref, wout_ref, bout_ref, out_ref,
                     emb_scr, bpad_scr, wtf_scr, wtb_scr):
    H = HID
    wtf_scr[...] = whhf_ref[...].T.astype(jnp.bfloat16)
    wtb_scr[...] = whhb_ref[...].T.astype(jnp.bfloat16)

    def cell(g, c):
        i = jax.nn.sigmoid(g[:, 0:H])
        f = jax.nn.sigmoid(g[:, H:2 * H])
        gg = jnp.tanh(g[:, 2 * H:3 * H])
        o = jax.nn.sigmoid(g[:, 3 * H:4 * H])
        c2 = f * c + i * gg
        h2 = o * jnp.tanh(c2)
        return h2, c2

    def step(t, carry):
        hf, cf, hb, cb = carry
        gf = xpf_ref[pl.ds(t, 1), :] + jnp.dot(
            hf.astype(jnp.bfloat16), wtf_scr[...],
            preferred_element_type=jnp.float32)
        gb = xpb_ref[pl.ds(S - 1 - t, 1), :] + jnp.dot(
            hb.astype(jnp.bfloat16), wtb_scr[...],
            preferred_element_type=jnp.float32)
        hf2, cf2 = cell(gf, cf)
        hb2, cb2 = cell(gb, cb)
        emb_scr[pl.ds(t, 1), 0:H] = hf2
        emb_scr[pl.ds(S - 1 - t, 1), H:2 * H] = hb2
        return hf2, cf2, hb2, cb2

    z = jnp.zeros((1, H), jnp.float32)
    jax.lax.fori_loop(0, S, step, (z, z, z, z), unroll=4)

    emb = emb_scr[...]
    A = jax.lax.dot_general(emb, wffn_ref[:, 0:2 * H], _DNT,
                            preferred_element_type=jnp.float32) + bffn_ref[...]
    B = jax.lax.dot_general(emb, wffn_ref[:, 2 * H:4 * H], _DNT,
                            preferred_element_type=jnp.float32)
    bpad_scr[0:S, :] = B
    bpad_scr[S:S + MPAD, :] = jnp.zeros((MPAD, HID), jnp.float32)
    rows = jax.lax.broadcasted_iota(jnp.int32, (S, 1), 0)
    for j in range(M):
        r = jnp.maximum(A + bpad_scr[j:j + S, :], 0.0)
        col = jnp.dot(r, wout_ref[...],
                      preferred_element_type=jnp.float32) + bout_ref[...]
        out_ref[:, j:j + 1] = jnp.where(rows + j < S, col, 0.0)


def kernel(subword_embeddings, W_ih_f, W_hh_f, b_ih_f, b_hh_f,
           W_ih_b, W_hh_b, b_ih_b, b_hh_b,
           W_ffnn, b_ffnn, W_out, b_out):
    f32 = jnp.float32
    bias_f = (b_ih_f + b_hh_f)[None, :]
    bias_b = (b_ih_b + b_hh_b)[None, :]

    RB = G // 2
    xpf, xpb = pl.pallas_call(
        _proj_kernel,
        grid=(G // RB,),
        in_specs=[
            pl.BlockSpec((S, D_IN), lambda i: (0, 0)),
            pl.BlockSpec((RB, D_IN), lambda i: (i, 0)),
            pl.BlockSpec((RB, D_IN), lambda i: (i, 0)),
            pl.BlockSpec((1, RB), lambda i: (0, i)),
            pl.BlockSpec((1, RB), lambda i: (0, i)),
        ],
        out_specs=[
            pl.BlockSpec((S, RB), lambda i: (0, i)),
            pl.BlockSpec((S, RB), lambda i: (0, i)),
        ],
        out_shape=[
            jax.ShapeDtypeStruct((S, G), f32),
            jax.ShapeDtypeStruct((S, G), f32),
        ],
    )(subword_embeddings, W_ih_f, W_ih_b, bias_f, bias_b)

    out2d = pl.pallas_call(
        _rec_span_kernel,
        out_shape=jax.ShapeDtypeStruct((S, M), f32),
        scratch_shapes=[
            pltpu.VMEM((S, 2 * HID), f32),
            pltpu.VMEM((S + MPAD, HID), f32),
            pltpu.VMEM((HID, G), jnp.bfloat16),
            pltpu.VMEM((HID, G), jnp.bfloat16),
        ],
    )(xpf, xpb, W_hh_f, W_hh_b, W_ffnn,
      b_ffnn[None, :], W_out.T, b_out[None, :])

    return out2d[:, :, None]


# speedup vs baseline: 26.4898x; 1.0426x over previous
import jax
import jax.numpy as jnp
from jax.experimental import pallas as pl
from jax.experimental.pallas import tpu as pltpu

S = 512
D_IN = 1536
HID = 384
M = 30
G = 4 * HID
MPAD = 32

_DNT = (((1,), (1,)), ((), ()))


def _proj_kernel(x_ref, wf_ref, wb_ref, bf_ref, bb_ref, of_ref, ob_ref):
    x = x_ref[...]
    of_ref[...] = jax.lax.dot_general(
        x, wf_ref[...], _DNT, preferred_element_type=jnp.float32) + bf_ref[...]
    ob_ref[...] = jax.lax.dot_general(
        x, wb_ref[...], _DNT, preferred_element_type=jnp.float32) + bb_ref[...]


def _rec_span_kernel(xpf_ref, xpb_ref, whhf_ref, whhb_ref, wffn_ref,
                     bffn_ref, wout_ref, bout_ref, out_ref,
                     emb_scr, bpad_scr, wtf_scr, wtb_scr):
    H = HID
    wtf_scr[...] = whhf_ref[...].T.astype(jnp.bfloat16)
    wtb_scr[...] = whhb_ref[...].T.astype(jnp.bfloat16)

    def cell(g, c):
        i = jax.nn.sigmoid(g[:, 0:H])
        f = jax.nn.sigmoid(g[:, H:2 * H])
        gg = jnp.tanh(g[:, 2 * H:3 * H])
        o = jax.nn.sigmoid(g[:, 3 * H:4 * H])
        c2 = f * c + i * gg
        h2 = o * jnp.tanh(c2)
        return h2, c2

    def step(t, carry):
        hf, cf, hb, cb = carry
        gf = xpf_ref[pl.ds(t, 1), :] + jnp.dot(
            hf.astype(jnp.bfloat16), wtf_scr[...],
            preferred_element_type=jnp.float32)
        gb = xpb_ref[pl.ds(S - 1 - t, 1), :] + jnp.dot(
            hb.astype(jnp.bfloat16), wtb_scr[...],
            preferred_element_type=jnp.float32)
        hf2, cf2 = cell(gf, cf)
        hb2, cb2 = cell(gb, cb)
        emb_scr[pl.ds(t, 1), 0:H] = hf2
        emb_scr[pl.ds(S - 1 - t, 1), H:2 * H] = hb2
        return hf2, cf2, hb2, cb2

    z = jnp.zeros((1, H), jnp.float32)
    jax.lax.fori_loop(0, S, step, (z, z, z, z), unroll=8)

    emb = emb_scr[...]
    A = jax.lax.dot_general(emb, wffn_ref[:, 0:2 * H], _DNT,
                            preferred_element_type=jnp.float32) + bffn_ref[...]
    B = jax.lax.dot_general(emb, wffn_ref[:, 2 * H:4 * H], _DNT,
                            preferred_element_type=jnp.float32)
    bpad_scr[0:S, :] = B
    bpad_scr[S:S + MPAD, :] = jnp.zeros((MPAD, HID), jnp.float32)
    rows = jax.lax.broadcasted_iota(jnp.int32, (S, 1), 0)
    for j in range(M):
        r = jnp.maximum(A + bpad_scr[j:j + S, :], 0.0)
        col = jnp.dot(r, wout_ref[...],
                      preferred_element_type=jnp.float32) + bout_ref[...]
        out_ref[:, j:j + 1] = jnp.where(rows + j < S, col, 0.0)


def kernel(subword_embeddings, W_ih_f, W_hh_f, b_ih_f, b_hh_f,
           W_ih_b, W_hh_b, b_ih_b, b_hh_b,
           W_ffnn, b_ffnn, W_out, b_out):
    f32 = jnp.float32
    bias_f = (b_ih_f + b_hh_f)[None, :]
    bias_b = (b_ih_b + b_hh_b)[None, :]

    RB = G // 2
    xpf, xpb = pl.pallas_call(
        _proj_kernel,
        grid=(G // RB,),
        in_specs=[
            pl.BlockSpec((S, D_IN), lambda i: (0, 0)),
            pl.BlockSpec((RB, D_IN), lambda i: (i, 0)),
            pl.BlockSpec((RB, D_IN), lambda i: (i, 0)),
            pl.BlockSpec((1, RB), lambda i: (0, i)),
            pl.BlockSpec((1, RB), lambda i: (0, i)),
        ],
        out_specs=[
            pl.BlockSpec((S, RB), lambda i: (0, i)),
            pl.BlockSpec((S, RB), lambda i: (0, i)),
        ],
        out_shape=[
            jax.ShapeDtypeStruct((S, G), f32),
            jax.ShapeDtypeStruct((S, G), f32),
        ],
    )(subword_embeddings, W_ih_f, W_ih_b, bias_f, bias_b)

    out2d = pl.pallas_call(
        _rec_span_kernel,
        out_shape=jax.ShapeDtypeStruct((S, M), f32),
        scratch_shapes=[
            pltpu.VMEM((S, 2 * HID), f32),
            pltpu.VMEM((S + MPAD, HID), f32),
            pltpu.VMEM((HID, G), jnp.bfloat16),
            pltpu.VMEM((HID, G), jnp.bfloat16),
        ],
    )(xpf, xpb, W_hh_f, W_hh_b, W_ffnn,
      b_ffnn[None, :], W_out.T, b_out[None, :])

    return out2d[:, :, None]


# speedup vs baseline: 34.5632x; 1.3048x over previous
import jax
import jax.numpy as jnp
from jax.experimental import pallas as pl
from jax.experimental.pallas import tpu as pltpu

S = 512
D_IN = 1536
HID = 384
M = 30
G = 4 * HID
MPAD = 32

_DNT = (((1,), (1,)), ((), ()))


def _fused_kernel(x_ref, wf_ref, wb_ref, bf_ref, bb_ref,
                  whhf_ref, whhb_ref, wffn_ref,
                  bffn_ref, wout_ref, bout_ref, out_ref,
                  xpf_scr, xpb_scr, emb_scr, bpad_scr,
                  wtf_scr, wtb_scr, wgf_scr, wgb_scr):
    H = HID
    x = x_ref[...]
    xpf_scr[...] = jax.lax.dot_general(
        x, wf_ref[...], _DNT, preferred_element_type=jnp.float32) + bf_ref[...]
    xpb_scr[...] = jax.lax.dot_general(
        x, wb_ref[...], _DNT, preferred_element_type=jnp.float32) + bb_ref[...]

    wtf_scr[:, 0:2 * H] = whhf_ref[0:2 * H, :].T.astype(jnp.float8_e4m3fn)
    wtf_scr[:, 2 * H:3 * H] = whhf_ref[3 * H:4 * H, :].T.astype(
        jnp.float8_e4m3fn)
    wtb_scr[:, 0:2 * H] = whhb_ref[0:2 * H, :].T.astype(jnp.float8_e4m3fn)
    wtb_scr[:, 2 * H:3 * H] = whhb_ref[3 * H:4 * H, :].T.astype(
        jnp.float8_e4m3fn)
    wgf_scr[...] = whhf_ref[2 * H:3 * H, :].T.astype(jnp.bfloat16)
    wgb_scr[...] = whhb_ref[2 * H:3 * H, :].T.astype(jnp.bfloat16)

    def halfstep(xp_row, h, c, wifo, wg):
        hb16 = h.astype(jnp.bfloat16)
        gifo = jnp.dot(hb16, wifo, preferred_element_type=jnp.float32)
        ggat = jnp.dot(hb16, wg, preferred_element_type=jnp.float32)
        i = jax.nn.sigmoid(xp_row[:, 0:H] + gifo[:, 0:H])
        f = jax.nn.sigmoid(xp_row[:, H:2 * H] + gifo[:, H:2 * H])
        o = jax.nn.sigmoid(xp_row[:, 3 * H:4 * H] + gifo[:, 2 * H:3 * H])
        gg = jnp.tanh(xp_row[:, 2 * H:3 * H] + ggat)
        c2 = f * c + i * gg
        h2 = o * jnp.tanh(c2)
        return h2, c2

    def step(t, carry):
        hf, cf, hb, cb = carry
        hf2, cf2 = halfstep(xpf_scr[pl.ds(t, 1), :], hf, cf,
                            wtf_scr[...], wgf_scr[...])
        hb2, cb2 = halfstep(xpb_scr[pl.ds(S - 1 - t, 1), :], hb, cb,
                            wtb_scr[...], wgb_scr[...])
        emb_scr[pl.ds(t, 1), 0:H] = hf2
        emb_scr[pl.ds(S - 1 - t, 1), H:2 * H] = hb2
        return hf2, cf2, hb2, cb2

    z = jnp.zeros((1, H), jnp.float32)
    jax.lax.fori_loop(0, S, step, (z, z, z, z), unroll=16)

    emb = emb_scr[...]
    A = jax.lax.dot_general(emb, wffn_ref[:, 0:2 * H], _DNT,
                            preferred_element_type=jnp.float32) + bffn_ref[...]
    B = jax.lax.dot_general(emb, wffn_ref[:, 2 * H:4 * H], _DNT,
                            preferred_element_type=jnp.float32)
    bpad_scr[0:S, :] = B
    bpad_scr[S:S + MPAD, :] = jnp.zeros((MPAD, HID), jnp.float32)
    rows = jax.lax.broadcasted_iota(jnp.int32, (S, 1), 0)
    for j in range(M):
        r = jnp.maximum(A + bpad_scr[j:j + S, :], 0.0)
        col = jnp.dot(r, wout_ref[...],
                      preferred_element_type=jnp.float32) + bout_ref[...]
        out_ref[:, j:j + 1] = jnp.where(rows + j < S, col, 0.0)


def kernel(subword_embeddings, W_ih_f, W_hh_f, b_ih_f, b_hh_f,
           W_ih_b, W_hh_b, b_ih_b, b_hh_b,
           W_ffnn, b_ffnn, W_out, b_out):
    f32 = jnp.float32
    bias_f = (b_ih_f + b_hh_f)[None, :]
    bias_b = (b_ih_b + b_hh_b)[None, :]

    out2d = pl.pallas_call(
        _fused_kernel,
        out_shape=jax.ShapeDtypeStruct((S, M), f32),
        scratch_shapes=[
            pltpu.VMEM((S, G), f32),
            pltpu.VMEM((S, G), f32),
            pltpu.VMEM((S, 2 * HID), f32),
            pltpu.VMEM((S + MPAD, HID), f32),
            pltpu.VMEM((HID, 3 * HID), jnp.float8_e4m3fn),
            pltpu.VMEM((HID, 3 * HID), jnp.float8_e4m3fn),
            pltpu.VMEM((HID, HID), jnp.bfloat16),
            pltpu.VMEM((HID, HID), jnp.bfloat16),
        ],
    )(subword_embeddings, W_ih_f, W_ih_b, bias_f, bias_b,
      W_hh_f, W_hh_b, W_ffnn,
      b_ffnn[None, :], W_out.T, b_out[None, :])

    return out2d[:, :, None]
